# Initial kernel scaffold; baseline (speedup 1.0000x reference)
import jax
import jax.numpy as jnp
from jax.experimental import pallas as pl


def kernel(x, W_g):
    raise NotImplementedError("write your pallas kernel here")


# speedup vs baseline: 1.0834x; 1.0834x over previous
import functools

import jax
import jax.numpy as jnp
from jax import lax
from jax.experimental import pallas as pl
from jax.experimental.pallas import tpu as pltpu

NUM_TOKENS = 16384
D_HIDDEN = 4096
NUM_EXPERTS = 64
TOP_K = 8
BLK = 512


def _gate_body(x_ref, w_ref, idx_ref, tks_ref, scores_ref):
    x = x_ref[...]
    w = w_ref[...]
    logits = lax.dot_general(
        x, w, (((1,), (1,)), ((), ())), preferred_element_type=jnp.float32
    )
    m = jnp.max(logits, axis=1, keepdims=True)
    e = jnp.exp(logits - m)
    s = jnp.sum(e, axis=1, keepdims=True)
    scores = e / s
    scores_ref[...] = scores

    work = scores
    iota = lax.broadcasted_iota(jnp.int32, (BLK, NUM_EXPERTS), 1)
    vals, idxs = [], []
    for _ in range(TOP_K):
        mx = jnp.max(work, axis=1, keepdims=True)
        hit = work == mx
        id_k = jnp.min(jnp.where(hit, iota, NUM_EXPERTS), axis=1, keepdims=True)
        vals.append(mx)
        idxs.append(id_k)
        work = jnp.where(iota == id_k, -1.0, work)
    v = jnp.concatenate(vals, axis=1)
    i = jnp.concatenate(idxs, axis=1)
    tks_ref[...] = v / jnp.sum(v, axis=1, keepdims=True)
    idx_ref[...] = i


def kernel(x, W_g):
    grid = (NUM_TOKENS // BLK,)
    out_shapes = (
        jax.ShapeDtypeStruct((NUM_TOKENS, TOP_K), jnp.int32),
        jax.ShapeDtypeStruct((NUM_TOKENS, TOP_K), jnp.float32),
        jax.ShapeDtypeStruct((NUM_TOKENS, NUM_EXPERTS), jnp.float32),
    )
    return pl.pallas_call(
        _gate_body,
        grid=grid,
        in_specs=[
            pl.BlockSpec((BLK, D_HIDDEN), lambda i: (i, 0)),
            pl.BlockSpec((NUM_EXPERTS, D_HIDDEN), lambda i: (0, 0)),
        ],
        out_specs=(
            pl.BlockSpec((BLK, TOP_K), lambda i: (i, 0)),
            pl.BlockSpec((BLK, TOP_K), lambda i: (i, 0)),
            pl.BlockSpec((BLK, NUM_EXPERTS), lambda i: (i, 0)),
        ),
        out_shape=out_shapes,
    )(x, W_g)
